# baseline (speedup 1.0000x reference)
import jax
import jax.numpy as jnp
from jax import lax
from jax.experimental import pallas as pl
from jax.experimental.pallas import tpu as pltpu
from jax.experimental.pallas import tpu_sc as plsc

D = 64
T = 200
B = 4096
V = 1000000
NC, NS = 2, 16
NW = NC * NS
LANES = 16

ROWS = B * T
ROWS_PER_W = ROWS // NW
CHUNK = 128
NCHUNK = ROWS_PER_W // CHUNK
POS_ROWS = 336
TBLK = 1024


def _relayout_body(src_ref, dst_ref):
    dst_ref[:, 0:D] = src_ref[...].T


def _emb_body(xw_hbm, tok_hbm, pos_hbm, out_hbm,
              idxw_v, wide_v, stage_v, pos_v, gsem, osem):
    wid = lax.axis_index("s") * NC + lax.axis_index("c")
    pltpu.sync_copy(pos_hbm, pos_v)
    row0 = wid * ROWS_PER_W
    ir0 = row0 // CHUNK

    def fetch(c, buf):
        pltpu.sync_copy(xw_hbm.at[ir0 + c], idxw_v.at[buf])
        pltpu.async_copy(tok_hbm.at[idxw_v.at[buf]], wide_v.at[buf], gsem)

    fetch(0, 0)

    def chunk_body(c, carry):
        buf = lax.rem(c, 2)
        pltpu.make_async_copy(tok_hbm.at[idxw_v.at[buf]], wide_v.at[buf],
                              gsem).wait()

        @pl.when(c + 1 < NCHUNK)
        def _():
            fetch(c + 1, 1 - buf)

        @pl.when(c >= 2)
        def _():
            pltpu.make_async_copy(
                stage_v.at[buf],
                out_hbm.at[pl.ds(row0 + (c - 2) * CHUNK, CHUNK)],
                osem,
            ).wait()

        phase = lax.rem(c * CHUNK, T)

        def group_body(g, carry2):
            base = g * LANES
            for i in range(LANES):
                r = base + i
                for j in range(D // LANES):
                    s = pl.ds(j * LANES, LANES)
                    stage_v[buf, r, s] = (
                        wide_v[buf, r, s] + pos_v[phase + r, s]
                    )
            return carry2

        lax.fori_loop(0, CHUNK // LANES, group_body, 0, unroll=False)

        pltpu.async_copy(
            stage_v.at[buf],
            out_hbm.at[pl.ds(row0 + c * CHUNK, CHUNK)],
            osem,
        )
        return carry

    lax.fori_loop(0, NCHUNK, chunk_body, 0, unroll=False)
    for c in (NCHUNK - 2, NCHUNK - 1):
        pltpu.make_async_copy(
            stage_v.at[c % 2],
            out_hbm.at[pl.ds(row0 + c * CHUNK, CHUNK)],
            osem,
        ).wait()


@jax.jit
def kernel(x, token_table, pos_table):
    xw = x.astype(jnp.int32).reshape(ROWS // CHUNK, CHUNK)
    tok2 = pl.pallas_call(
        _relayout_body,
        grid=(pl.cdiv(V, TBLK),),
        in_specs=[pl.BlockSpec((D, TBLK), lambda j: (0, j))],
        out_specs=pl.BlockSpec((TBLK, 128), lambda j: (j, 0)),
        out_shape=jax.ShapeDtypeStruct((V, 128), jnp.float32),
    )(token_table.T)
    rr = jnp.arange(POS_ROWS) % T
    pos2 = jnp.pad(pos_table[rr], ((0, 0), (0, 128 - D)))
    mesh = plsc.VectorSubcoreMesh(core_axis_name="c", subcore_axis_name="s")
    run = pl.kernel(
        _emb_body,
        mesh=mesh,
        out_type=jax.ShapeDtypeStruct((ROWS, D), jnp.float32),
        scratch_types=[
            pltpu.VMEM((2, CHUNK), jnp.int32),
            pltpu.VMEM((2, CHUNK, 128), jnp.float32),
            pltpu.VMEM((2, CHUNK, D), jnp.float32),
            pltpu.VMEM((POS_ROWS, 128), jnp.float32),
            pltpu.SemaphoreType.DMA,
            pltpu.SemaphoreType.DMA,
        ],
    )
    out = run(xw, tok2, pos2)
    return out.reshape(B, T, D)


# speedup vs baseline: 1.0454x; 1.0454x over previous
import jax
import jax.numpy as jnp
from jax import lax
from jax.experimental import pallas as pl
from jax.experimental.pallas import tpu as pltpu
from jax.experimental.pallas import tpu_sc as plsc

D = 64
T = 200
B = 4096
V = 1000000
NC, NS = 2, 16
NW = NC * NS
LANES = 16

ROWS = B * T
ROWS_PER_W = ROWS // NW
CHUNK = 128
NCHUNK = ROWS_PER_W // CHUNK
NBUF = 3
TBLK = 1024


def _relayout_body(src_ref, dst_ref):
    eye = (lax.broadcasted_iota(jnp.int32, (D, D), 0)
           == lax.broadcasted_iota(jnp.int32, (D, D), 1)).astype(jnp.float32)
    dst_ref[:, 0:D] = lax.dot_general(
        src_ref[...], eye,
        dimension_numbers=(((0,), (0,)), ((), ())),
        preferred_element_type=jnp.float32,
    )


def _emb_body(xw_hbm, tok_hbm, pos_hbm, out_hbm,
              idx_v, wide_v, stage_v, pos_v, gsem, osem):
    wid = lax.axis_index("s") * NC + lax.axis_index("c")
    pltpu.sync_copy(xw_hbm.at[pl.ds(wid * NCHUNK, NCHUNK)], idx_v)
    pltpu.sync_copy(pos_hbm, pos_v)
    row0 = wid * ROWS_PER_W

    def fire(c, slot):
        pltpu.async_copy(tok_hbm.at[idx_v.at[c]], wide_v.at[slot],
                         gsem.at[slot])

    for c in range(NBUF):
        fire(c, c)

    def chunk_body(c, carry):
        slot = lax.rem(c, NBUF)
        sslot = lax.rem(c, 2)
        pltpu.make_async_copy(tok_hbm.at[idx_v.at[c]], wide_v.at[slot],
                              gsem.at[slot]).wait()

        @pl.when(c >= 2)
        def _():
            pltpu.make_async_copy(
                stage_v.at[sslot],
                out_hbm.at[pl.ds(row0 + (c - 2) * CHUNK, CHUNK)],
                osem.at[sslot],
            ).wait()

        phase = lax.rem(c * CHUNK, T)

        def group_body(g, carry2):
            base = g * LANES
            for i in range(LANES):
                r = base + i
                pr = phase + r
                pr = lax.select(pr >= T, pr - T, pr)
                ph, po = pr >> 1, (pr & 1) * D
                for j in range(D // LANES):
                    s = pl.ds(j * LANES, LANES)
                    stage_v[sslot, r, s] = (
                        wide_v[slot, r, s]
                        + pos_v[ph, pl.ds(po + j * LANES, LANES)]
                    )
            return carry2

        lax.fori_loop(0, CHUNK // LANES, group_body, 0, unroll=False)

        pltpu.async_copy(
            stage_v.at[sslot],
            out_hbm.at[pl.ds(row0 + c * CHUNK, CHUNK)],
            osem.at[sslot],
        )

        @pl.when(c + NBUF < NCHUNK)
        def _():
            fire(c + NBUF, slot)

        return carry

    lax.fori_loop(0, NCHUNK, chunk_body, 0, unroll=False)
    for c in (NCHUNK - 2, NCHUNK - 1):
        pltpu.make_async_copy(
            stage_v.at[c % 2],
            out_hbm.at[pl.ds(row0 + c * CHUNK, CHUNK)],
            osem.at[c % 2],
        ).wait()


@jax.jit
def kernel(x, token_table, pos_table):
    xw = x.astype(jnp.int32).reshape(ROWS // CHUNK, CHUNK)
    tok2 = pl.pallas_call(
        _relayout_body,
        grid=(pl.cdiv(V, TBLK),),
        in_specs=[pl.BlockSpec((D, TBLK), lambda j: (0, j))],
        out_specs=pl.BlockSpec((TBLK, 128), lambda j: (j, 0)),
        out_shape=jax.ShapeDtypeStruct((V, 128), jnp.float32),
    )(token_table.T)
    pos2 = pos_table.reshape(T // 2, 128)
    mesh = plsc.VectorSubcoreMesh(core_axis_name="c", subcore_axis_name="s")
    run = pl.kernel(
        _emb_body,
        mesh=mesh,
        out_type=jax.ShapeDtypeStruct((ROWS, D), jnp.float32),
        scratch_types=[
            pltpu.VMEM((NCHUNK, CHUNK), jnp.int32),
            pltpu.VMEM((NBUF, CHUNK, 128), jnp.float32),
            pltpu.VMEM((2, CHUNK, D), jnp.float32),
            pltpu.VMEM((T // 2, 128), jnp.float32),
            pltpu.SemaphoreType.DMA((NBUF,)),
            pltpu.SemaphoreType.DMA((2,)),
        ],
    )
    out = run(xw, tok2, pos2)
    return out.reshape(B, T, D)
